# baseline (speedup 1.0000x reference)
import functools
import math

import jax
import jax.numpy as jnp
from jax import lax
from jax.experimental import pallas as pl
from jax.experimental.pallas import tpu as pltpu
from jax.experimental.pallas import tpu_sc as plsc

D_MODEL = 64
B = 4096
L = 200
NW = 32
BPW = B // NW
HALF = 100
NH = BPW * (L // HALF)
SCALE = math.sqrt(float(D_MODEL))

_mesh = plsc.VectorSubcoreMesh(core_axis_name="c", subcore_axis_name="s")


@functools.partial(
    pl.kernel,
    mesh=_mesh,
    compiler_params=pltpu.CompilerParams(use_tc_tiling_on_sc=False),
    out_type=jax.ShapeDtypeStruct((B * L // HALF, HALF, D_MODEL), jnp.float32),
    scratch_types=[
        pltpu.VMEM((NH, HALF), jnp.int32),
        pltpu.VMEM((BPW,), jnp.int32),
        pltpu.VMEM((BPW, D_MODEL), jnp.float32),
        pltpu.VMEM((2, HALF, D_MODEL), jnp.float32),
        pltpu.SemaphoreType.DMA,
    ],
)
def _embed_sc(task0_hbm, uni_hbm, ttab_hbm, utab_hbm, out_hbm,
              uidx_v, tidx_v, trows_v, buf_v, sem):
    wid = lax.axis_index("s") * 2 + lax.axis_index("c")
    base_b = wid * BPW

    pltpu.sync_copy(uni_hbm.at[pl.ds(wid * NH, NH)], uidx_v)
    pltpu.sync_copy(task0_hbm.at[pl.ds(base_b, BPW)], tidx_v)
    pltpu.async_copy(ttab_hbm.at[tidx_v], trows_v, sem).wait()

    def step(j, carry):
        cp0 = pltpu.async_copy(utab_hbm.at[uidx_v.at[2 * j]], buf_v.at[0], sem)
        cp1 = pltpu.async_copy(utab_hbm.at[uidx_v.at[2 * j + 1]], buf_v.at[1], sem)
        cp0.wait()
        cp1.wait()

        def scale_row(r, c):
            for h in range(2):
                for k in range(D_MODEL // 16):
                    sl = pl.ds(k * 16, 16)
                    buf_v[h, r, sl] = buf_v[h, r, sl] * SCALE
            return c

        lax.fori_loop(0, HALF, scale_row, 0)
        for k in range(D_MODEL // 16):
            sl = pl.ds(k * 16, 16)
            buf_v[0, 0, sl] = trows_v[j, sl] * SCALE
        pltpu.sync_copy(buf_v, out_hbm.at[pl.ds((base_b + j) * 2, 2)])
        return carry

    lax.fori_loop(0, BPW, step, 0)


def kernel(task, uni, task_table, uni_table):
    task0 = task[:, 0]
    uni_r = uni.reshape(B * L // HALF, HALF)
    out = _embed_sc(task0, uni_r, task_table, uni_table)
    return out.reshape(B, L, D_MODEL)
